# Initial kernel scaffold; baseline (speedup 1.0000x reference)
import jax
import jax.numpy as jnp
from jax.experimental import pallas as pl


def kernel(inputs, grid, Wpos, bpos, g_enc, b_enc, Wm1, bm1, Wm2, bm2, Wa1, ba1, Wa2, ba2, Wq, bq, Wk, bk, Wv, bv, Wih, Whh, bih, bhh, Wf1, bf1, Wf2, bf2, g_in, b_in, g_sl, b_sl, g_ff, b_ff, emb):
    raise NotImplementedError("write your pallas kernel here")


# speedup vs baseline: 1.2707x; 1.2707x over previous
import jax
import jax.numpy as jnp
from jax.experimental import pallas as pl
from jax.experimental.pallas import tpu as pltpu

B = 16
H = 32
W = 32
D = 256
N = H * W
S = 64
K = 8192
HID = 256
ITERS = 3
EPS = 1e-8
BETA = 0.99
SCALE = D ** (-0.5)
BS = B * S
CH = 1024
F32 = jnp.float32

_HI = jax.lax.Precision.HIGHEST


def _dot(a, b):
    return jnp.dot(a, b, preferred_element_type=F32)


def _dgen(a, b, dims):
    return jax.lax.dot_general(a, b, (dims, ((), ())),
                               preferred_element_type=F32)


def _dot_hi(a, b):
    return jnp.dot(a, b, preferred_element_type=F32, precision=_HI)


def _dgen_hi(a, b, dims):
    return jax.lax.dot_general(a, b, (dims, ((), ())),
                               preferred_element_type=F32, precision=_HI)


def _pos_body(x_ref, g_ref, Wpos_ref, bpos_ref, s1_ref):
    s1_ref[0] = x_ref[0] + (_dot(g_ref[...], Wpos_ref[...]) + bpos_ref[...])


def _enc_body(x_ref, mv_ref, g_enc_ref, b_enc_ref,
              Wm1_ref, bm1_ref, Wm2_ref, bm2_ref, Wa1_ref, ba1_ref,
              Wa2s_ref, ba2c_ref, g_in_ref, b_in_ref,
              Wk_ref, bk_ref, Wv_ref, bv_ref,
              anch_ref, k_ref, v_ref):
    m = mv_ref[0, 0, 0]
    var = mv_ref[0, 0, 1]
    x = (x_ref[0] - m) / jnp.sqrt(var + 1e-5) * g_enc_ref[...] + b_enc_ref[...]
    h = jnp.maximum(_dot(x, Wm1_ref[...]) + bm1_ref[...], 0.0)
    h = _dot(h, Wm2_ref[...]) + bm2_ref[...]
    a1 = _dgen(h, Wa1_ref[...], (((0,), (0,))))
    a1 = jnp.maximum(a1 + ba1_ref[...], 0.0)
    anch = _dgen(Wa2s_ref[...], a1, (((0,), (1,))))
    anch_ref[0] = anch + ba2c_ref[...]
    mr = jnp.mean(h, axis=1, keepdims=True)
    vr = jnp.mean((h - mr) ** 2, axis=1, keepdims=True)
    hn = (h - mr) / jnp.sqrt(vr + 1e-5) * g_in_ref[...] + b_in_ref[...]
    k_ref[0] = _dot(hn, Wk_ref[...]) + bk_ref[...]
    v_ref[0] = _dot(hn, Wv_ref[...]) + bv_ref[...]


def _vq_body(zf_ref, emb_ref, Wq_ref, bq_ref,
             zq_ref, fidx_ref, loss_ref):
    zf = zf_ref[...]
    zn = jnp.sum(zf * zf, axis=1, keepdims=True)
    ones_row = jnp.ones((1, D), F32)
    NCH = K // CH

    def chunk(c, carry):
        gmin, gidx = carry
        ec = emb_ref[pl.ds(c * CH, CH), :]
        ts = _dot(ec, Wq_ref[...]) + bq_ref[...]
        tn = _dgen_hi(ones_row, ts * ts, (((1,), (1,))))
        cross = _dgen(zf, ts, (((1,), (1,))))
        dc = zn + tn - 2.0 * cross
        mval = jnp.min(dc, axis=1, keepdims=True)
        col = jax.lax.broadcasted_iota(jnp.int32, (BS, CH), 1).astype(F32)
        marg = jnp.min(jnp.where(dc == mval, col, F32(K)), axis=1,
                       keepdims=True) + (c * CH).astype(F32)
        better = mval < gmin
        return jnp.where(better, mval, gmin), jnp.where(better, marg, gidx)

    gmin, gidx = jax.lax.fori_loop(
        0, NCH, chunk,
        (jnp.full((BS, 1), jnp.inf, F32), jnp.zeros((BS, 1), F32)))

    ri = jax.lax.broadcasted_iota(jnp.int32, (BS, BS), 0)
    ci = jax.lax.broadcasted_iota(jnp.int32, (BS, BS), 1)
    A = (ri // S == ci // S).astype(F32)
    qi = jax.lax.broadcasted_iota(jnp.int32, (BS, S), 0)
    ti = jax.lax.broadcasted_iota(jnp.int32, (BS, S), 1)
    diag = (qi % S == ti)
    Z = _dot_hi(A, jnp.where(diag, gmin, 0.0))
    sp = qi % S
    cmp = (Z < gmin) | ((Z == gmin) & (ti < sp))
    r = jnp.sum(cmp.astype(F32), axis=1, keepdims=True)
    R2 = _dot_hi(A, jnp.where(diag, r, 0.0))
    P = (R2 == sp.astype(F32)).astype(F32)

    idx0 = gidx[0:S, :]
    fidx_ref[...] = _dot_hi(P, idx0).astype(jnp.int32)
    kio = jax.lax.broadcasted_iota(jnp.int32, (S, K), 1).astype(F32)
    OH = (kio == idx0).astype(F32)
    z0 = _dot_hi(OH, emb_ref[...])
    zq = _dot_hi(P, z0)
    zq_ref[...] = zq
    tzq = _dot(zq, Wq_ref[...]) + bq_ref[...]
    diff = tzq - zf
    loss_ref[...] = jnp.sum(diff * diff, keepdims=True) * ((1.0 + BETA) / B)


def _attn_body(z_ref, k_ref, v_ref, Wq_ref, bq_ref, g_sl_ref, b_sl_ref,
               Wih_ref, Whh_ref, bih_ref, bhh_ref, g_ff_ref, b_ff_ref,
               Wf1_ref, bf1_ref, Wf2_ref, bf2_ref, out_ref):
    slots = z_ref[0]
    k = k_ref[0]
    v = v_ref[0]
    for _ in range(ITERS):
        prev = slots
        mu = jnp.mean(slots, axis=1, keepdims=True)
        var = jnp.mean((slots - mu) ** 2, axis=1, keepdims=True)
        sn = (slots - mu) / jnp.sqrt(var + 1e-5) * g_sl_ref[...] + b_sl_ref[...]
        q = _dot(sn, Wq_ref[...]) + bq_ref[...]
        dots = _dgen(q, k, (((1,), (1,)))) * SCALE
        mx = jnp.max(dots, axis=0, keepdims=True)
        e = jnp.exp(dots - mx)
        attn = e / jnp.sum(e, axis=0, keepdims=True) + EPS
        attn = attn / jnp.sum(attn, axis=1, keepdims=True)
        upd = _dot(attn, v)
        gi = _dot(upd, Wih_ref[...]) + bih_ref[...]
        gh = _dot(prev, Whh_ref[...]) + bhh_ref[...]
        rg = jax.nn.sigmoid(gi[:, :D] + gh[:, :D])
        zg = jax.nn.sigmoid(gi[:, D:2 * D] + gh[:, D:2 * D])
        ng = jnp.tanh(gi[:, 2 * D:] + rg * gh[:, 2 * D:])
        slots = (1.0 - zg) * ng + zg * prev
        mu2 = jnp.mean(slots, axis=1, keepdims=True)
        var2 = jnp.mean((slots - mu2) ** 2, axis=1, keepdims=True)
        fn = (slots - mu2) / jnp.sqrt(var2 + 1e-5) * g_ff_ref[...] + b_ff_ref[...]
        ff = jnp.maximum(_dot(fn, Wf1_ref[...]) + bf1_ref[...], 0.0)
        slots = slots + _dot(ff, Wf2_ref[...]) + bf2_ref[...]
    out_ref[0] = slots


def _full(shape):
    n = len(shape)
    return pl.BlockSpec(shape, lambda b, _n=n: (0,) * _n)


def kernel(inputs, grid, Wpos, bpos, g_enc, b_enc, Wm1, bm1, Wm2, bm2,
           Wa1, ba1, Wa2, ba2, Wq, bq, Wk, bk, Wv, bv,
           Wih, Whh, bih, bhh, Wf1, bf1, Wf2, bf2,
           g_in, b_in, g_sl, b_sl, g_ff, b_ff, emb):
    x = inputs.reshape(B, N, D)
    gridr = grid.reshape(N, 4)
    row = lambda a: a.reshape(1, -1)
    Wa2s = Wa2[:, :S]
    ba2c = ba2[:S].reshape(S, 1)

    s1 = pl.pallas_call(
        _pos_body,
        grid=(B,),
        in_specs=[
            pl.BlockSpec((1, N, D), lambda b: (b, 0, 0)),
            _full((N, 4)), _full((4, D)), _full((1, D)),
        ],
        out_specs=pl.BlockSpec((1, N, D), lambda b: (b, 0, 0)),
        out_shape=jax.ShapeDtypeStruct((B, N, D), F32),
    )(x, gridr, Wpos, row(bpos))

    xs = (inputs + (grid @ Wpos + bpos)).reshape(B, N, D)
    m = jnp.mean(xs, axis=(-2, -1), keepdims=True)
    v = jnp.mean((xs - m) ** 2, axis=(-2, -1), keepdims=True)
    mv = jnp.concatenate([m.reshape(B, 1), v.reshape(B, 1)], axis=1)
    mv = mv.reshape(B, 1, 2)

    anch, kk, vv = pl.pallas_call(
        _enc_body,
        grid=(B,),
        in_specs=[
            pl.BlockSpec((1, N, D), lambda b: (b, 0, 0)),
            pl.BlockSpec((1, 1, 2), lambda b: (b, 0, 0), memory_space=pltpu.SMEM),
            _full((N, D)), _full((N, D)),
            _full((D, D)), _full((1, D)), _full((D, D)), _full((1, D)),
            _full((N, N)), _full((1, N)),
            _full((N, S)), _full((S, 1)),
            _full((1, D)), _full((1, D)),
            _full((D, D)), _full((1, D)), _full((D, D)), _full((1, D)),
        ],
        out_specs=[
            pl.BlockSpec((1, S, D), lambda b: (b, 0, 0)),
            pl.BlockSpec((1, N, D), lambda b: (b, 0, 0)),
            pl.BlockSpec((1, N, D), lambda b: (b, 0, 0)),
        ],
        out_shape=[
            jax.ShapeDtypeStruct((B, S, D), F32),
            jax.ShapeDtypeStruct((B, N, D), F32),
            jax.ShapeDtypeStruct((B, N, D), F32),
        ],
    )(s1, mv, g_enc, b_enc, Wm1, row(bm1), Wm2, row(bm2),
      Wa1, row(ba1), Wa2s, ba2c, row(g_in), row(b_in),
      Wk, row(bk), Wv, row(bv))

    zf = anch.reshape(BS, D)
    zq, fidx, loss = pl.pallas_call(
        _vq_body,
        out_shape=[
            jax.ShapeDtypeStruct((BS, D), F32),
            jax.ShapeDtypeStruct((BS, 1), jnp.int32),
            jax.ShapeDtypeStruct((1, 1), F32),
        ],
    )(zf, emb, Wq, row(bq))

    slots = pl.pallas_call(
        _attn_body,
        grid=(B,),
        in_specs=[
            pl.BlockSpec((1, S, D), lambda b: (b, 0, 0)),
            pl.BlockSpec((1, N, D), lambda b: (b, 0, 0)),
            pl.BlockSpec((1, N, D), lambda b: (b, 0, 0)),
            _full((D, D)), _full((1, D)), _full((1, D)), _full((1, D)),
            _full((D, 3 * D)), _full((D, 3 * D)), _full((1, 3 * D)), _full((1, 3 * D)),
            _full((1, D)), _full((1, D)),
            _full((D, HID)), _full((1, HID)), _full((HID, D)), _full((1, D)),
        ],
        out_specs=pl.BlockSpec((1, S, D), lambda b: (b, 0, 0)),
        out_shape=jax.ShapeDtypeStruct((B, S, D), F32),
    )(zq.reshape(B, S, D), kk, vv, Wq, row(bq), row(g_sl), row(b_sl),
      Wih, Whh, row(bih), row(bhh), row(g_ff), row(b_ff),
      Wf1, row(bf1), Wf2, row(bf2))

    return slots, loss.reshape(()), fidx.reshape(B, S)


# speedup vs baseline: 1.3644x; 1.0737x over previous
import jax
import jax.numpy as jnp
from jax.experimental import pallas as pl
from jax.experimental.pallas import tpu as pltpu

B = 16
H = 32
W = 32
D = 256
N = H * W
S = 64
K = 8192
HID = 256
ITERS = 3
EPS = 1e-8
BETA = 0.99
SCALE = D ** (-0.5)
BS = B * S
CH = 1024
F32 = jnp.float32

_HI = jax.lax.Precision.HIGHEST


def _dot(a, b):
    return jnp.dot(a, b, preferred_element_type=F32)


def _dgen(a, b, dims):
    return jax.lax.dot_general(a, b, (dims, ((), ())),
                               preferred_element_type=F32)


def _dot_hi(a, b):
    return jnp.dot(a, b, preferred_element_type=F32, precision=_HI)


def _dgen_hi(a, b, dims):
    return jax.lax.dot_general(a, b, (dims, ((), ())),
                               preferred_element_type=F32, precision=_HI)


def _enc_body(x_ref, g_ref, Wpos_ref, bpos_ref, mv_ref, g_enc_ref, b_enc_ref,
              Wm1_ref, bm1_ref, Wm2_ref, bm2_ref, Wa1_ref, ba1_ref,
              Wa2s_ref, ba2c_ref, g_in_ref, b_in_ref,
              Wk_ref, bk_ref, Wv_ref, bv_ref,
              anch_ref, k_ref, v_ref):
    s1 = x_ref[0] + (_dot(g_ref[...], Wpos_ref[...]) + bpos_ref[...])
    m = mv_ref[0, 0, 0]
    var = mv_ref[0, 0, 1]
    x = (s1 - m) / jnp.sqrt(var + 1e-5) * g_enc_ref[...] + b_enc_ref[...]
    h = jnp.maximum(_dot(x, Wm1_ref[...]) + bm1_ref[...], 0.0)
    h = _dot(h, Wm2_ref[...]) + bm2_ref[...]
    a1 = _dgen(h, Wa1_ref[...], (((0,), (0,))))
    a1 = jnp.maximum(a1 + ba1_ref[...], 0.0)
    anch = _dgen(Wa2s_ref[...], a1, (((0,), (1,))))
    anch_ref[0] = anch + ba2c_ref[...]
    mr = jnp.mean(h, axis=1, keepdims=True)
    vr = jnp.mean((h - mr) ** 2, axis=1, keepdims=True)
    hn = (h - mr) / jnp.sqrt(vr + 1e-5) * g_in_ref[...] + b_in_ref[...]
    k_ref[0] = _dot(hn, Wk_ref[...]) + bk_ref[...]
    v_ref[0] = _dot(hn, Wv_ref[...]) + bv_ref[...]


def _vq_body(zf_ref, emb_ref, Wq_ref, bq_ref,
             zq_ref, fidx_ref, loss_ref):
    zf = zf_ref[...]
    zn = jnp.sum(zf * zf, axis=1, keepdims=True)
    ones_row = jnp.ones((1, D), F32)
    NCH = K // CH

    def chunk(c, carry):
        gmin, gidx = carry
        ec = emb_ref[pl.ds(c * CH, CH), :]
        ts = _dot(ec, Wq_ref[...]) + bq_ref[...]
        tn = _dgen_hi(ones_row, ts * ts, (((1,), (1,))))
        cross = _dgen(zf, ts, (((1,), (1,))))
        dc = zn + tn - 2.0 * cross
        mval = jnp.min(dc, axis=1, keepdims=True)
        col = jax.lax.broadcasted_iota(jnp.int32, (BS, CH), 1).astype(F32)
        marg = jnp.min(jnp.where(dc == mval, col, F32(K)), axis=1,
                       keepdims=True) + (c * CH).astype(F32)
        better = mval < gmin
        return jnp.where(better, mval, gmin), jnp.where(better, marg, gidx)

    gmin, gidx = jax.lax.fori_loop(
        0, NCH, chunk,
        (jnp.full((BS, 1), jnp.inf, F32), jnp.zeros((BS, 1), F32)))

    ri = jax.lax.broadcasted_iota(jnp.int32, (BS, BS), 0)
    ci = jax.lax.broadcasted_iota(jnp.int32, (BS, BS), 1)
    A = (ri // S == ci // S).astype(F32)
    qi = jax.lax.broadcasted_iota(jnp.int32, (BS, S), 0)
    ti = jax.lax.broadcasted_iota(jnp.int32, (BS, S), 1)
    diag = (qi % S == ti)
    Z = _dot_hi(A, jnp.where(diag, gmin, 0.0))
    sp = qi % S
    cmp = (Z < gmin) | ((Z == gmin) & (ti < sp))
    r = jnp.sum(cmp.astype(F32), axis=1, keepdims=True)
    R2 = _dot_hi(A, jnp.where(diag, r, 0.0))
    P = (R2 == sp.astype(F32)).astype(F32)

    idx0 = gidx[0:S, :]
    fidx_ref[...] = _dot_hi(P, idx0).astype(jnp.int32)
    kio = jax.lax.broadcasted_iota(jnp.int32, (S, K), 1).astype(F32)
    OH = (kio == idx0).astype(F32)
    z0 = _dot_hi(OH, emb_ref[...])
    zq = _dot_hi(P, z0)
    zq_ref[...] = zq
    tzq = _dot(zq, Wq_ref[...]) + bq_ref[...]
    diff = tzq - zf
    loss_ref[...] = jnp.sum(diff * diff, keepdims=True) * ((1.0 + BETA) / B)


def _attn_body(z_ref, k_ref, v_ref, Wq_ref, bq_ref, g_sl_ref, b_sl_ref,
               Wih_ref, Whh_ref, bih_ref, bhh_ref, g_ff_ref, b_ff_ref,
               Wf1_ref, bf1_ref, Wf2_ref, bf2_ref, out_ref):
    slots = z_ref[0]
    k = k_ref[0]
    v = v_ref[0]
    for _ in range(ITERS):
        prev = slots
        mu = jnp.mean(slots, axis=1, keepdims=True)
        var = jnp.mean((slots - mu) ** 2, axis=1, keepdims=True)
        sn = (slots - mu) / jnp.sqrt(var + 1e-5) * g_sl_ref[...] + b_sl_ref[...]
        q = _dot(sn, Wq_ref[...]) + bq_ref[...]
        dots = _dgen(q, k, (((1,), (1,)))) * SCALE
        mx = jnp.max(dots, axis=0, keepdims=True)
        e = jnp.exp(dots - mx)
        attn = e / jnp.sum(e, axis=0, keepdims=True) + EPS
        attn = attn / jnp.sum(attn, axis=1, keepdims=True)
        upd = _dot(attn, v)
        gi = _dot(upd, Wih_ref[...]) + bih_ref[...]
        gh = _dot(prev, Whh_ref[...]) + bhh_ref[...]
        rg = jax.nn.sigmoid(gi[:, :D] + gh[:, :D])
        zg = jax.nn.sigmoid(gi[:, D:2 * D] + gh[:, D:2 * D])
        ng = jnp.tanh(gi[:, 2 * D:] + rg * gh[:, 2 * D:])
        slots = (1.0 - zg) * ng + zg * prev
        mu2 = jnp.mean(slots, axis=1, keepdims=True)
        var2 = jnp.mean((slots - mu2) ** 2, axis=1, keepdims=True)
        fn = (slots - mu2) / jnp.sqrt(var2 + 1e-5) * g_ff_ref[...] + b_ff_ref[...]
        ff = jnp.maximum(_dot(fn, Wf1_ref[...]) + bf1_ref[...], 0.0)
        slots = slots + _dot(ff, Wf2_ref[...]) + bf2_ref[...]
    out_ref[0] = slots


def _full(shape):
    n = len(shape)
    return pl.BlockSpec(shape, lambda b, _n=n: (0,) * _n)


def kernel(inputs, grid, Wpos, bpos, g_enc, b_enc, Wm1, bm1, Wm2, bm2,
           Wa1, ba1, Wa2, ba2, Wq, bq, Wk, bk, Wv, bv,
           Wih, Whh, bih, bhh, Wf1, bf1, Wf2, bf2,
           g_in, b_in, g_sl, b_sl, g_ff, b_ff, emb):
    x = inputs.reshape(B, N, D)
    gridr = grid.reshape(N, 4)
    row = lambda a: a.reshape(1, -1)
    Wa2s = Wa2[:, :S]
    ba2c = ba2[:S].reshape(S, 1)

    xs = (inputs + (grid @ Wpos + bpos)).reshape(B, N, D)
    m = jnp.mean(xs, axis=(-2, -1), keepdims=True)
    v = jnp.mean((xs - m) ** 2, axis=(-2, -1), keepdims=True)
    mv = jnp.concatenate([m.reshape(B, 1), v.reshape(B, 1)], axis=1)
    mv = mv.reshape(B, 1, 2)

    anch, kk, vv = pl.pallas_call(
        _enc_body,
        grid=(B,),
        compiler_params=pltpu.CompilerParams(
            dimension_semantics=("parallel",)),
        in_specs=[
            pl.BlockSpec((1, N, D), lambda b: (b, 0, 0)),
            _full((N, 4)), _full((4, D)), _full((1, D)),
            pl.BlockSpec((1, 1, 2), lambda b: (b, 0, 0), memory_space=pltpu.SMEM),
            _full((N, D)), _full((N, D)),
            _full((D, D)), _full((1, D)), _full((D, D)), _full((1, D)),
            _full((N, N)), _full((1, N)),
            _full((N, S)), _full((S, 1)),
            _full((1, D)), _full((1, D)),
            _full((D, D)), _full((1, D)), _full((D, D)), _full((1, D)),
        ],
        out_specs=[
            pl.BlockSpec((1, S, D), lambda b: (b, 0, 0)),
            pl.BlockSpec((1, N, D), lambda b: (b, 0, 0)),
            pl.BlockSpec((1, N, D), lambda b: (b, 0, 0)),
        ],
        out_shape=[
            jax.ShapeDtypeStruct((B, S, D), F32),
            jax.ShapeDtypeStruct((B, N, D), F32),
            jax.ShapeDtypeStruct((B, N, D), F32),
        ],
    )(x, gridr, Wpos, row(bpos), mv, g_enc, b_enc, Wm1, row(bm1), Wm2, row(bm2),
      Wa1, row(ba1), Wa2s, ba2c, row(g_in), row(b_in),
      Wk, row(bk), Wv, row(bv))

    zf = anch.reshape(BS, D)
    zq, fidx, loss = pl.pallas_call(
        _vq_body,
        out_shape=[
            jax.ShapeDtypeStruct((BS, D), F32),
            jax.ShapeDtypeStruct((BS, 1), jnp.int32),
            jax.ShapeDtypeStruct((1, 1), F32),
        ],
    )(zf, emb, Wq, row(bq))

    slots = pl.pallas_call(
        _attn_body,
        grid=(B,),
        compiler_params=pltpu.CompilerParams(
            dimension_semantics=("parallel",)),
        in_specs=[
            pl.BlockSpec((1, S, D), lambda b: (b, 0, 0)),
            pl.BlockSpec((1, N, D), lambda b: (b, 0, 0)),
            pl.BlockSpec((1, N, D), lambda b: (b, 0, 0)),
            _full((D, D)), _full((1, D)), _full((1, D)), _full((1, D)),
            _full((D, 3 * D)), _full((D, 3 * D)), _full((1, 3 * D)), _full((1, 3 * D)),
            _full((1, D)), _full((1, D)),
            _full((D, HID)), _full((1, HID)), _full((HID, D)), _full((1, D)),
        ],
        out_specs=pl.BlockSpec((1, S, D), lambda b: (b, 0, 0)),
        out_shape=jax.ShapeDtypeStruct((B, S, D), F32),
    )(zq.reshape(B, S, D), kk, vv, Wq, row(bq), row(g_sl), row(b_sl),
      Wih, Whh, row(bih), row(bhh), row(g_ff), row(b_ff),
      Wf1, row(bf1), Wf2, row(bf2))

    return slots, loss.reshape(()), fidx.reshape(B, S)


# speedup vs baseline: 1.3914x; 1.0197x over previous
import jax
import jax.numpy as jnp
from jax.experimental import pallas as pl
from jax.experimental.pallas import tpu as pltpu

B = 16
H = 32
W = 32
D = 256
N = H * W
S = 64
K = 8192
HID = 256
ITERS = 3
EPS = 1e-8
BETA = 0.99
SCALE = D ** (-0.5)
BS = B * S
CH = 2048
F32 = jnp.float32

_HI = jax.lax.Precision.HIGHEST


def _dot(a, b):
    return jnp.dot(a, b, preferred_element_type=F32)


def _dgen(a, b, dims):
    return jax.lax.dot_general(a, b, (dims, ((), ())),
                               preferred_element_type=F32)


def _dot_hi(a, b):
    return jnp.dot(a, b, preferred_element_type=F32, precision=_HI)


def _dgen_hi(a, b, dims):
    return jax.lax.dot_general(a, b, (dims, ((), ())),
                               preferred_element_type=F32, precision=_HI)


def _enc_body(x_ref, g_ref, Wpos_ref, bpos_ref, mv_ref, g_enc_ref, b_enc_ref,
              Wm1_ref, bm1_ref, Wm2_ref, bm2_ref, Wa1_ref, ba1_ref,
              Wa2s_ref, ba2c_ref, g_in_ref, b_in_ref,
              Wk_ref, bk_ref, Wv_ref, bv_ref,
              anch_ref, k_ref, v_ref):
    s1 = x_ref[0] + (_dot(g_ref[...], Wpos_ref[...]) + bpos_ref[...])
    m = mv_ref[0, 0, 0]
    var = mv_ref[0, 0, 1]
    x = (s1 - m) / jnp.sqrt(var + 1e-5) * g_enc_ref[...] + b_enc_ref[...]
    h = jnp.maximum(_dot(x, Wm1_ref[...]) + bm1_ref[...], 0.0)
    h = _dot(h, Wm2_ref[...]) + bm2_ref[...]
    a1 = _dgen(h, Wa1_ref[...], (((0,), (0,))))
    a1 = jnp.maximum(a1 + ba1_ref[...], 0.0)
    anch = _dgen(Wa2s_ref[...], a1, (((0,), (1,))))
    anch_ref[0] = anch + ba2c_ref[...]
    mr = jnp.mean(h, axis=1, keepdims=True)
    vr = jnp.mean((h - mr) ** 2, axis=1, keepdims=True)
    hn = (h - mr) / jnp.sqrt(vr + 1e-5) * g_in_ref[...] + b_in_ref[...]
    k_ref[0] = _dot(hn, Wk_ref[...]) + bk_ref[...]
    v_ref[0] = _dot(hn, Wv_ref[...]) + bv_ref[...]


def _vq_body(zf_ref, emb_ref, Wq_ref, bq_ref,
             zq_ref, fidx_ref, loss_ref):
    zf = zf_ref[...]
    zn = jnp.sum(zf * zf, axis=1, keepdims=True)
    ones_row = jnp.ones((1, D), F32)
    NCH = K // CH

    def chunk(c, carry):
        gmin, gidx = carry
        ec = emb_ref[pl.ds(c * CH, CH), :]
        ts = _dot(ec, Wq_ref[...]) + bq_ref[...]
        tn = _dgen_hi(ones_row, ts * ts, (((1,), (1,))))
        cross = _dgen(zf, ts, (((1,), (1,))))
        dc = zn + tn - 2.0 * cross
        mval = jnp.min(dc, axis=1, keepdims=True)
        col = jax.lax.broadcasted_iota(jnp.int32, (BS, CH), 1).astype(F32)
        marg = jnp.min(jnp.where(dc == mval, col, F32(K)), axis=1,
                       keepdims=True) + (c * CH).astype(F32)
        better = mval < gmin
        return jnp.where(better, mval, gmin), jnp.where(better, marg, gidx)

    gmin, gidx = jax.lax.fori_loop(
        0, NCH, chunk,
        (jnp.full((BS, 1), jnp.inf, F32), jnp.zeros((BS, 1), F32)))

    ri = jax.lax.broadcasted_iota(jnp.int32, (BS, BS), 0)
    ci = jax.lax.broadcasted_iota(jnp.int32, (BS, BS), 1)
    A = (ri // S == ci // S).astype(F32)
    qi = jax.lax.broadcasted_iota(jnp.int32, (BS, S), 0)
    ti = jax.lax.broadcasted_iota(jnp.int32, (BS, S), 1)
    diag = (qi % S == ti)
    Z = _dot_hi(A, jnp.where(diag, gmin, 0.0))
    sp = qi % S
    cmp = (Z < gmin) | ((Z == gmin) & (ti < sp))
    r = jnp.sum(cmp.astype(F32), axis=1, keepdims=True)
    R2 = _dot_hi(A, jnp.where(diag, r, 0.0))
    P = (R2 == sp.astype(F32)).astype(F32)

    idx0 = gidx[0:S, :]
    fidx_ref[...] = _dot_hi(P, idx0).astype(jnp.int32)
    kio = jax.lax.broadcasted_iota(jnp.int32, (S, K), 1).astype(F32)
    OH = (kio == idx0).astype(F32)
    z0 = _dot_hi(OH, emb_ref[...])
    zq = _dot_hi(P, z0)
    zq_ref[...] = zq
    tzq = _dot(zq, Wq_ref[...]) + bq_ref[...]
    diff = tzq - zf
    loss_ref[...] = jnp.sum(diff * diff, keepdims=True) * ((1.0 + BETA) / B)


def _attn_body(z_ref, k_ref, v_ref, Wq_ref, bq_ref, g_sl_ref, b_sl_ref,
               Wih_ref, Whh_ref, bih_ref, bhh_ref, g_ff_ref, b_ff_ref,
               Wf1_ref, bf1_ref, Wf2_ref, bf2_ref, out_ref):
    slots = z_ref[0]
    k = k_ref[0]
    v = v_ref[0]
    for _ in range(ITERS):
        prev = slots
        mu = jnp.mean(slots, axis=1, keepdims=True)
        var = jnp.mean((slots - mu) ** 2, axis=1, keepdims=True)
        sn = (slots - mu) / jnp.sqrt(var + 1e-5) * g_sl_ref[...] + b_sl_ref[...]
        q = _dot(sn, Wq_ref[...]) + bq_ref[...]
        dots = _dgen(q, k, (((1,), (1,)))) * SCALE
        mx = jnp.max(dots, axis=0, keepdims=True)
        e = jnp.exp(dots - mx)
        attn = e / jnp.sum(e, axis=0, keepdims=True) + EPS
        attn = attn / jnp.sum(attn, axis=1, keepdims=True)
        upd = _dot(attn, v)
        gi = _dot(upd, Wih_ref[...]) + bih_ref[...]
        gh = _dot(prev, Whh_ref[...]) + bhh_ref[...]
        rg = jax.nn.sigmoid(gi[:, :D] + gh[:, :D])
        zg = jax.nn.sigmoid(gi[:, D:2 * D] + gh[:, D:2 * D])
        ng = jnp.tanh(gi[:, 2 * D:] + rg * gh[:, 2 * D:])
        slots = (1.0 - zg) * ng + zg * prev
        mu2 = jnp.mean(slots, axis=1, keepdims=True)
        var2 = jnp.mean((slots - mu2) ** 2, axis=1, keepdims=True)
        fn = (slots - mu2) / jnp.sqrt(var2 + 1e-5) * g_ff_ref[...] + b_ff_ref[...]
        ff = jnp.maximum(_dot(fn, Wf1_ref[...]) + bf1_ref[...], 0.0)
        slots = slots + _dot(ff, Wf2_ref[...]) + bf2_ref[...]
    out_ref[0] = slots


def _full(shape):
    n = len(shape)
    return pl.BlockSpec(shape, lambda b, _n=n: (0,) * _n)


def kernel(inputs, grid, Wpos, bpos, g_enc, b_enc, Wm1, bm1, Wm2, bm2,
           Wa1, ba1, Wa2, ba2, Wq, bq, Wk, bk, Wv, bv,
           Wih, Whh, bih, bhh, Wf1, bf1, Wf2, bf2,
           g_in, b_in, g_sl, b_sl, g_ff, b_ff, emb):
    x = inputs.reshape(B, N, D)
    gridr = grid.reshape(N, 4)
    row = lambda a: a.reshape(1, -1)
    Wa2s = Wa2[:, :S]
    ba2c = ba2[:S].reshape(S, 1)

    xs = (inputs + (grid @ Wpos + bpos)).reshape(B, N, D)
    m = jnp.mean(xs, axis=(-2, -1), keepdims=True)
    v = jnp.mean((xs - m) ** 2, axis=(-2, -1), keepdims=True)
    mv = jnp.concatenate([m.reshape(B, 1), v.reshape(B, 1)], axis=1)
    mv = mv.reshape(B, 1, 2)

    anch, kk, vv = pl.pallas_call(
        _enc_body,
        grid=(B,),
        compiler_params=pltpu.CompilerParams(
            dimension_semantics=("parallel",)),
        in_specs=[
            pl.BlockSpec((1, N, D), lambda b: (b, 0, 0)),
            _full((N, 4)), _full((4, D)), _full((1, D)),
            pl.BlockSpec((1, 1, 2), lambda b: (b, 0, 0), memory_space=pltpu.SMEM),
            _full((N, D)), _full((N, D)),
            _full((D, D)), _full((1, D)), _full((D, D)), _full((1, D)),
            _full((N, N)), _full((1, N)),
            _full((N, S)), _full((S, 1)),
            _full((1, D)), _full((1, D)),
            _full((D, D)), _full((1, D)), _full((D, D)), _full((1, D)),
        ],
        out_specs=[
            pl.BlockSpec((1, S, D), lambda b: (b, 0, 0)),
            pl.BlockSpec((1, N, D), lambda b: (b, 0, 0)),
            pl.BlockSpec((1, N, D), lambda b: (b, 0, 0)),
        ],
        out_shape=[
            jax.ShapeDtypeStruct((B, S, D), F32),
            jax.ShapeDtypeStruct((B, N, D), F32),
            jax.ShapeDtypeStruct((B, N, D), F32),
        ],
    )(x, gridr, Wpos, row(bpos), mv, g_enc, b_enc, Wm1, row(bm1), Wm2, row(bm2),
      Wa1, row(ba1), Wa2s, ba2c, row(g_in), row(b_in),
      Wk, row(bk), Wv, row(bv))

    zf = anch.reshape(BS, D)
    zq, fidx, loss = pl.pallas_call(
        _vq_body,
        out_shape=[
            jax.ShapeDtypeStruct((BS, D), F32),
            jax.ShapeDtypeStruct((BS, 1), jnp.int32),
            jax.ShapeDtypeStruct((1, 1), F32),
        ],
    )(zf, emb, Wq, row(bq))

    slots = pl.pallas_call(
        _attn_body,
        grid=(B,),
        compiler_params=pltpu.CompilerParams(
            dimension_semantics=("parallel",)),
        in_specs=[
            pl.BlockSpec((1, S, D), lambda b: (b, 0, 0)),
            pl.BlockSpec((1, N, D), lambda b: (b, 0, 0)),
            pl.BlockSpec((1, N, D), lambda b: (b, 0, 0)),
            _full((D, D)), _full((1, D)), _full((1, D)), _full((1, D)),
            _full((D, 3 * D)), _full((D, 3 * D)), _full((1, 3 * D)), _full((1, 3 * D)),
            _full((1, D)), _full((1, D)),
            _full((D, HID)), _full((1, HID)), _full((HID, D)), _full((1, D)),
        ],
        out_specs=pl.BlockSpec((1, S, D), lambda b: (b, 0, 0)),
        out_shape=jax.ShapeDtypeStruct((B, S, D), F32),
    )(zq.reshape(B, S, D), kk, vv, Wq, row(bq), row(g_sl), row(b_sl),
      Wih, Whh, row(bih), row(bhh), row(g_ff), row(b_ff),
      Wf1, row(bf1), Wf2, row(bf2))

    return slots, loss.reshape(()), fidx.reshape(B, S)
